# baseline (speedup 1.0000x reference)
import jax
import jax.numpy as jnp
from jax import lax
from jax.experimental import pallas as pl
from jax.experimental.pallas import tpu as pltpu

N_VIEW_BLOCK = 512

_DNUMS = (((1,), (1,)), ((), ()))


def _fused_embed_kernel(
    x_ref, ws_ref, bs_ref, wr_ref, br_ref, src_ref, rx_ref, w4s_ref, w4r_ref, b4_ref
):
    @pl.when(pl.program_id(0) == 0)
    def _assemble():
        w4s_ref[...] = jnp.zeros((128, 512), jnp.float32)
        w4r_ref[...] = jnp.zeros((128, 512), jnp.float32)
        for k in range(4):
            w4s_ref[pl.ds(32 * k, 32), pl.ds(128 * k, 128)] = ws_ref[...]
            w4r_ref[pl.ds(32 * k, 32), pl.ds(128 * k, 128)] = wr_ref[...]
            b4_ref[0, pl.ds(32 * k, 32)] = bs_ref[0, :]
            b4_ref[1, pl.ds(32 * k, 32)] = br_ref[0, :]

    x4 = x_ref[...]
    src_ref[...] = lax.dot_general(
        x4, w4s_ref[...], _DNUMS, preferred_element_type=jnp.float32
    ) + b4_ref[0, :][None, :]
    rx_ref[...] = lax.dot_general(
        x4, w4r_ref[...], _DNUMS, preferred_element_type=jnp.float32
    ) + b4_ref[1, :][None, :]


@jax.jit
def kernel(x, edge_index, W_src, b_src, W_rx, b_rx):
    del edge_index
    n, c = x.shape
    nv = n // 4
    x4 = x.reshape(nv, 4 * c)
    grid = -(-nv // N_VIEW_BLOCK)
    src, rx = pl.pallas_call(
        _fused_embed_kernel,
        grid=(grid,),
        in_specs=[
            pl.BlockSpec((N_VIEW_BLOCK, 4 * c), lambda i: (i, 0)),
            pl.BlockSpec((32, c), lambda i: (0, 0)),
            pl.BlockSpec((1, 32), lambda i: (0, 0)),
            pl.BlockSpec((32, c), lambda i: (0, 0)),
            pl.BlockSpec((1, 32), lambda i: (0, 0)),
        ],
        out_specs=[
            pl.BlockSpec((N_VIEW_BLOCK, 128), lambda i: (i, 0)),
            pl.BlockSpec((N_VIEW_BLOCK, 128), lambda i: (i, 0)),
        ],
        out_shape=[
            jax.ShapeDtypeStruct((nv, 128), jnp.float32),
            jax.ShapeDtypeStruct((nv, 128), jnp.float32),
        ],
        scratch_shapes=[
            pltpu.VMEM((128, 512), jnp.float32),
            pltpu.VMEM((128, 512), jnp.float32),
            pltpu.VMEM((8, 128), jnp.float32),
        ],
        compiler_params=pltpu.CompilerParams(
            dimension_semantics=("arbitrary",),
        ),
    )(x4, W_src, b_src[None, :], W_rx, b_rx[None, :])
    return (src.reshape(n, 32), rx.reshape(n, 32))


# speedup vs baseline: 1.5858x; 1.5858x over previous
import jax
import jax.numpy as jnp
from jax import lax
from jax.experimental import pallas as pl
from jax.experimental.pallas import tpu as pltpu

N_ROWS_PER_BLOCK = 5000

_DNUMS = (((1,), (1,)), ((), ()))


def _fused_embed_kernel(x_ref, ws_ref, bs_ref, wr_ref, br_ref, src_ref, rx_ref):
    x = x_ref[...]
    src_ref[...] = lax.dot_general(
        x, ws_ref[...], _DNUMS, preferred_element_type=jnp.float32
    ) + bs_ref[...]
    rx_ref[...] = lax.dot_general(
        x, wr_ref[...], _DNUMS, preferred_element_type=jnp.float32
    ) + br_ref[...]


@jax.jit
def kernel(x, edge_index, W_src, b_src, W_rx, b_rx):
    del edge_index
    n, k = x.shape
    grid = n // N_ROWS_PER_BLOCK
    src, rx = pl.pallas_call(
        _fused_embed_kernel,
        grid=(grid,),
        in_specs=[
            pl.BlockSpec((N_ROWS_PER_BLOCK, k), lambda i: (i, 0)),
            pl.BlockSpec((32, k), lambda i: (0, 0)),
            pl.BlockSpec((1, 32), lambda i: (0, 0)),
            pl.BlockSpec((32, k), lambda i: (0, 0)),
            pl.BlockSpec((1, 32), lambda i: (0, 0)),
        ],
        out_specs=[
            pl.BlockSpec((N_ROWS_PER_BLOCK, 32), lambda i: (i, 0)),
            pl.BlockSpec((N_ROWS_PER_BLOCK, 32), lambda i: (i, 0)),
        ],
        out_shape=[
            jax.ShapeDtypeStruct((n, 32), jnp.float32),
            jax.ShapeDtypeStruct((n, 32), jnp.float32),
        ],
        compiler_params=pltpu.CompilerParams(
            dimension_semantics=("arbitrary",),
        ),
    )(x, W_src, b_src[None, :], W_rx, b_rx[None, :])
    return (src, rx)
